# baseline (speedup 1.0000x reference)
import functools
import jax
import jax.numpy as jnp
from jax import lax
from jax.experimental import pallas as pl
from jax.experimental.pallas import tpu as pltpu
from jax.experimental.pallas import tpu_sc as plsc

N = 10000
E = 320000
D = 128

NC = 2
NS = 16
NW = NC * NS
K = 128
W_CHUNKS = 80
EPW = W_CHUNKS * K
NPAD = 10240
NPW = NPAD // NS
ZR = 16

_MESH = plsc.VectorSubcoreMesh(core_axis_name="c", subcore_axis_name="s")


def _zero_shared(acc, zbuf, sub):
    def zero_row(r, carry):
        for cc in range(D // 16):
            zbuf[r, pl.ds(cc * 16, 16)] = jnp.zeros((16,), jnp.float32)
        return carry

    lax.fori_loop(0, ZR, zero_row, 0)
    for j in range(NPW // ZR):
        pltpu.sync_copy(zbuf, acc.at[pl.ds(sub * NPW + j * ZR, ZR)])


def _deg_body(eidx_hbm, out_hbm, idx_v, ones_v, zbuf, acc):
    core = lax.axis_index("c")
    sub = lax.axis_index("s")
    wid = core * NS + sub

    def ones_row(r, carry):
        for cc in range(D // 16):
            ones_v[r, pl.ds(cc * 16, 16)] = jnp.ones((16,), jnp.float32)
        return carry

    lax.fori_loop(0, K, ones_row, 0)
    _zero_shared(acc, zbuf, sub)
    plsc.subcore_barrier()

    pltpu.sync_copy(eidx_hbm.at[wid], idx_v)

    def chunk(j, carry):
        pltpu.sync_copy(ones_v, acc.at[idx_v.at[j, 1]], add=True)
        return carry

    lax.fori_loop(0, W_CHUNKS, chunk, 0)
    plsc.subcore_barrier()
    pltpu.sync_copy(acc.at[pl.ds(sub * NPW, NPW)], out_hbm.at[core, sub])


@functools.partial(
    pl.kernel,
    out_type=jax.ShapeDtypeStruct((NC, NS, NPW, D), jnp.float32),
    mesh=_MESH,
    scratch_types=[
        pltpu.VMEM((W_CHUNKS, 2, K), jnp.int32),
        pltpu.VMEM((K, D), jnp.float32),
        pltpu.VMEM((ZR, D), jnp.float32),
        pltpu.VMEM_SHARED((NPAD, D), jnp.float32),
    ],
)
def _deg_kernel(eidx_hbm, out_hbm, idx_v, ones_v, zbuf, acc):
    _deg_body(eidx_hbm, out_hbm, idx_v, ones_v, zbuf, acc)


def _edge_body(g_hbm, eidx_hbm, out_hbm,
               ib0, ib1, ib2, ib3, rows0, rows1, zbuf, acc,
               si0, si1, si2, si3, sr0, sr1):
    core = lax.axis_index("c")
    sub = lax.axis_index("s")
    wid = core * NS + sub

    _zero_shared(acc, zbuf, sub)
    plsc.subcore_barrier()

    ibs = [ib0, ib1, ib2, ib3]
    sis = [si0, si1, si2, si3]
    rows = [rows0, rows1]
    srs = [sr0, sr1]

    def idx_load(j, c):
        pltpu.async_copy(eidx_hbm.at[wid, j], ibs[c], sis[c])

    def idx_wait(c):
        pltpu.make_async_copy(eidx_hbm.at[wid, 0], ibs[c], sis[c]).wait()

    def gather(c, p):
        pltpu.async_copy(g_hbm.at[ibs[c].at[0]], rows[p], srs[p])

    def gather_wait(p):
        pltpu.make_async_copy(g_hbm.at[ibs[0].at[0]], rows[p], srs[p]).wait()

    def scatter(p, c):
        pltpu.sync_copy(rows[p], acc.at[ibs[c].at[1]], add=True)

    for c in range(4):
        idx_load(c, c)
    idx_wait(0)
    gather(0, 0)

    def quad(i, carry):
        a = 4 * i
        for c in range(4):
            j = a + c

            @pl.when(j + 1 < W_CHUNKS)
            def _():
                idx_wait((c + 1) % 4)
                gather((c + 1) % 4, (c + 1) % 2)

            gather_wait(c % 2)
            scatter(c % 2, c)

            @pl.when(j + 4 < W_CHUNKS)
            def _():
                idx_load(j + 4, c)

        return carry

    lax.fori_loop(0, W_CHUNKS // 4, quad, 0)

    plsc.subcore_barrier()
    pltpu.sync_copy(acc.at[pl.ds(sub * NPW, NPW)], out_hbm.at[core, sub])


@functools.partial(
    pl.kernel,
    out_type=jax.ShapeDtypeStruct((NC, NS, NPW, D), jnp.float32),
    mesh=_MESH,
    scratch_types=[
        pltpu.VMEM((2, K), jnp.int32),
        pltpu.VMEM((2, K), jnp.int32),
        pltpu.VMEM((2, K), jnp.int32),
        pltpu.VMEM((2, K), jnp.int32),
        pltpu.VMEM((K, D), jnp.float32),
        pltpu.VMEM((K, D), jnp.float32),
        pltpu.VMEM((ZR, D), jnp.float32),
        pltpu.VMEM_SHARED((NPAD, D), jnp.float32),
        pltpu.SemaphoreType.DMA,
        pltpu.SemaphoreType.DMA,
        pltpu.SemaphoreType.DMA,
        pltpu.SemaphoreType.DMA,
        pltpu.SemaphoreType.DMA,
        pltpu.SemaphoreType.DMA,
    ],
)
def _edge_kernel(g_hbm, eidx_hbm, out_hbm,
                 ib0, ib1, ib2, ib3, rows0, rows1, zbuf, acc,
                 si0, si1, si2, si3, sr0, sr1):
    _edge_body(g_hbm, eidx_hbm, out_hbm,
               ib0, ib1, ib2, ib3, rows0, rows1, zbuf, acc,
               si0, si1, si2, si3, sr0, sr1)


BR = 1000


def _mm_body(x_ref, w_ref, out_ref):
    out_ref[...] = jnp.dot(x_ref[...], w_ref[...],
                           preferred_element_type=jnp.float32)


def _tc_mm(x, w):
    return pl.pallas_call(
        _mm_body,
        grid=(N // BR,),
        in_specs=[
            pl.BlockSpec((BR, D), lambda i: (i, 0)),
            pl.BlockSpec((D, D), lambda i: (0, 0)),
        ],
        out_specs=pl.BlockSpec((BR, D), lambda i: (i, 0)),
        out_shape=jax.ShapeDtypeStruct((N, D), jnp.float32),
    )(x, w)


def _scale_body(d0_ref, d1_ref, h_ref, dinv_ref, g1_ref):
    deg = d0_ref[...] + d1_ref[...] + 1.0
    dinv = lax.rsqrt(jnp.maximum(deg, 1.0))
    dinv_ref[...] = dinv
    g1_ref[...] = h_ref[...] * dinv


def _tc_scale(deg, h):
    return pl.pallas_call(
        _scale_body,
        grid=(N // BR,),
        in_specs=[
            pl.BlockSpec((BR, D), lambda i: (i, 0)),
            pl.BlockSpec((BR, D), lambda i: (i, 0)),
            pl.BlockSpec((BR, D), lambda i: (i, 0)),
        ],
        out_specs=[
            pl.BlockSpec((BR, D), lambda i: (i, 0)),
            pl.BlockSpec((BR, D), lambda i: (i, 0)),
        ],
        out_shape=[
            jax.ShapeDtypeStruct((N, D), jnp.float32),
            jax.ShapeDtypeStruct((N, D), jnp.float32),
        ],
    )(deg[0], deg[1], h)


def _tc2_body(p0_ref, p1_ref, g_ref, dinv_ref, b_ref, w_ref, out_ref):
    agg = p0_ref[...] + p1_ref[...] + g_ref[...]
    xn = jax.nn.relu(agg * dinv_ref[...] + b_ref[...])
    h = jnp.dot(xn, w_ref[...], preferred_element_type=jnp.float32)
    out_ref[...] = h * dinv_ref[...]


def _tc3_body(p0_ref, p1_ref, g_ref, dinv_ref, b_ref, w_ref, b3_ref, out_ref):
    agg = p0_ref[...] + p1_ref[...] + g_ref[...]
    xn = jax.nn.relu(agg * dinv_ref[...] + b_ref[...])
    h = jnp.dot(xn, w_ref[...], preferred_element_type=jnp.float32)
    out_ref[...] = h + b3_ref[...]


def _tc_stage2(p, g, dinv, b, w):
    return pl.pallas_call(
        _tc2_body,
        grid=(N // BR,),
        in_specs=[
            pl.BlockSpec((BR, D), lambda i: (i, 0)),
            pl.BlockSpec((BR, D), lambda i: (i, 0)),
            pl.BlockSpec((BR, D), lambda i: (i, 0)),
            pl.BlockSpec((BR, D), lambda i: (i, 0)),
            pl.BlockSpec((1, D), lambda i: (0, 0)),
            pl.BlockSpec((D, D), lambda i: (0, 0)),
        ],
        out_specs=pl.BlockSpec((BR, D), lambda i: (i, 0)),
        out_shape=jax.ShapeDtypeStruct((N, D), jnp.float32),
    )(p[0], p[1], g, dinv, b.reshape(1, D), w)


def _tc_stage3(p, g, dinv, b, w, b3):
    return pl.pallas_call(
        _tc3_body,
        grid=(N // BR,),
        in_specs=[
            pl.BlockSpec((BR, D), lambda i: (i, 0)),
            pl.BlockSpec((BR, D), lambda i: (i, 0)),
            pl.BlockSpec((BR, D), lambda i: (i, 0)),
            pl.BlockSpec((BR, D), lambda i: (i, 0)),
            pl.BlockSpec((1, D), lambda i: (0, 0)),
            pl.BlockSpec((D, D), lambda i: (0, 0)),
            pl.BlockSpec((1, D), lambda i: (0, 0)),
        ],
        out_specs=pl.BlockSpec((BR, D), lambda i: (i, 0)),
        out_shape=jax.ShapeDtypeStruct((N, D), jnp.float32),
    )(p[0], p[1], g, dinv, b.reshape(1, D), w, b3.reshape(1, D))


@jax.jit
def kernel(x, edge_index, W1, b1, W2, b2, W3, b3):
    pad_per_w = EPW - E // NW
    src_p = jnp.concatenate(
        [edge_index[0].reshape(NW, E // NW),
         jnp.zeros((NW, pad_per_w), jnp.int32)], axis=1)
    dst_p = jnp.concatenate(
        [edge_index[1].reshape(NW, E // NW),
         jnp.full((NW, pad_per_w), NPAD - 1, jnp.int32)], axis=1)
    eidx = jnp.stack([src_p.reshape(NW, W_CHUNKS, K),
                      dst_p.reshape(NW, W_CHUNKS, K)], axis=2)

    deg = _deg_kernel(eidx).reshape(NC, NPAD, D)
    h1 = _tc_mm(x, W1)
    dinv, g1 = _tc_scale(deg, h1)
    p1 = _edge_kernel(g1, eidx).reshape(NC, NPAD, D)
    g2 = _tc_stage2(p1, g1, dinv, b1, W2)
    p2 = _edge_kernel(g2, eidx).reshape(NC, NPAD, D)
    return _tc_stage3(p2, g2, dinv, b2, W3, b3)
